# baseline (speedup 1.0000x reference)
import functools

import jax
import jax.numpy as jnp
from jax import lax
from jax.experimental import pallas as pl
from jax.experimental.pallas import tpu as pltpu, tpu_sc as plsc

E = 8
D = 1024
DFF = 2048
N = 2048
TILE = 256
T = N // TILE
WL = T + E

_LR = 16
_LC = 128


def _route_body(lab_ref, pos_ref, meta_ref):
    lab = lab_ref[...]

    ci = lax.broadcasted_iota(jnp.int32, (_LC, _LC), 0)
    cj = lax.broadcasted_iota(jnp.int32, (_LC, _LC), 1)
    u_lane = (ci <= cj).astype(jnp.float32)
    ri = lax.broadcasted_iota(jnp.int32, (_LR, _LR), 0)
    rj = lax.broadcasted_iota(jnp.int32, (_LR, _LR), 1)
    l_row = (rj < ri).astype(jnp.float32)

    pos_f = jnp.zeros((_LR, _LC), jnp.float32)
    starts, ends = [], []
    run = jnp.zeros((), jnp.float32)
    for e in range(E):
        m = (lab == e).astype(jnp.float32)
        within = jnp.dot(m, u_lane, preferred_element_type=jnp.float32)
        rowsum = jnp.sum(m, axis=1, keepdims=True)
        rowpref = jnp.dot(l_row, rowsum, preferred_element_type=jnp.float32)
        rank = within + rowpref
        cnt = jnp.sum(rowsum)
        starts.append(run)
        run = run + cnt
        ends.append(run)
        pos_f = pos_f + m * (starts[e] + rank - 1.0)
    pos_ref[...] = pos_f.astype(jnp.int32)

    one = jnp.ones((), jnp.float32)
    zero = jnp.zeros((), jnp.float32)
    tlos, nspans = [], []
    woff = [zero]
    for e in range(E):
        cnt = ends[e] - starts[e]
        nonempty = (cnt > 0).astype(jnp.float32)
        tlo = jnp.floor(starts[e] / TILE)
        thi = jnp.floor((ends[e] - one) / TILE)
        n = nonempty * (thi - tlo + one)
        tlos.append(tlo)
        nspans.append(n)
        woff.append(woff[-1] + n)
    total = woff[-1]

    ts, es, los, his = [], [], [], []
    for w in range(WL):
        wf = jnp.minimum(jnp.float32(w), total - one)
        t_w = zero
        e_w = zero
        s_w = zero
        en_w = zero
        for e in range(E):
            inside = ((woff[e] <= wf) & (wf < woff[e + 1])).astype(jnp.float32)
            t_e = tlos[e] + (wf - woff[e])
            t_w = t_w + inside * t_e
            e_w = e_w + inside * e
            s_w = s_w + inside * starts[e]
            en_w = en_w + inside * ends[e]
        valid = (jnp.float32(w) < total).astype(jnp.float32)
        ts.append(t_w)
        es.append(e_w)
        los.append(valid * jnp.maximum(s_w - t_w * TILE, zero))
        his.append(valid * jnp.minimum(en_w - t_w * TILE, jnp.float32(TILE)))

    firsts, runs = [], []
    runidx = zero
    for w in range(WL):
        if w == 0:
            f = one
        else:
            f = (es[w] != es[w - 1]).astype(jnp.float32)
            runidx = runidx + f
        firsts.append(f)
        runs.append(runidx)
    nxts, hasns, nxt2s, hasn2s = [], [], [], []
    for w in range(WL):
        nxt = es[w]
        nxt2 = es[w]
        hasn = zero
        hasn2 = zero
        for v in range(WL - 1, w, -1):
            take = firsts[v]
            nxt2 = take * nxt + (one - take) * nxt2
            hasn2 = take * hasn + (one - take) * hasn2
            nxt = take * es[v] + (one - take) * nxt
            hasn = jnp.maximum(hasn, take)
        nxts.append(nxt)
        hasns.append(hasn)
        nxt2s.append(nxt2)
        hasn2s.append(hasn2)

    iota_w = lax.broadcasted_iota(jnp.int32, (1, WL), 1)
    rows_vals = [ts, es, los, his, firsts, runs, nxts, hasns, nxt2s, hasn2s]
    rows = []
    for vals in rows_vals:
        row = jnp.zeros((1, WL), jnp.float32)
        for w in range(WL):
            row = row + (iota_w == w).astype(jnp.float32) * vals[w]
        rows.append(row)
    meta_ref[...] = jnp.concatenate(rows, axis=0).astype(jnp.int32)


def _ffn_body(m_ref, x_ref, b1_ref, b2_ref, w1_hbm, w2_hbm, o_ref,
              w1_buf, w2_buf, sems):
    w = pl.program_id(0)
    e = m_ref[1, w]
    lo = m_ref[2, w]
    hi = m_ref[3, w]
    first = m_ref[4, w]
    run = m_ref[5, w]
    nxt = m_ref[6, w]
    hasn = m_ref[7, w]
    nxt2 = m_ref[8, w]
    hasn2 = m_ref[9, w]
    slot = lax.rem(run, 3)

    def w_copies(exp, s):
        return (
            pltpu.make_async_copy(w1_hbm.at[exp], w1_buf.at[s], sems.at[0, s]),
            pltpu.make_async_copy(w2_hbm.at[exp], w2_buf.at[s], sems.at[1, s]),
        )

    @pl.when(w == 0)
    def _():
        for c in w_copies(e, slot):
            c.start()

        @pl.when(hasn == 1)
        def _():
            for c in w_copies(nxt, lax.rem(run + 1, 3)):
                c.start()

    @pl.when(first == 1)
    def _():
        @pl.when(hasn2 == 1)
        def _():
            for c in w_copies(nxt2, lax.rem(run + 2, 3)):
                c.start()

        for c in w_copies(e, slot):
            c.wait()

    @pl.when(hi > lo)
    def _():
        x = x_ref[...]
        h = jnp.dot(x, w1_buf[slot], preferred_element_type=jnp.float32)
        h = h + b1_ref[0]
        h = jax.nn.gelu(h)
        y = jnp.dot(h, w2_buf[slot], preferred_element_type=jnp.float32)
        y = y + b2_ref[0]
        rows = lax.broadcasted_iota(jnp.int32, (TILE, 1), 0)
        mask = (rows >= lo) & (rows < hi)
        o_ref[...] = jnp.where(mask, y, o_ref[...])


_NC = 2
_NW = 32
_CHUNK = N // _NW


@functools.lru_cache(maxsize=None)
def _sc_kernels():
    mesh = plsc.VectorSubcoreMesh(core_axis_name="c", subcore_axis_name="s")
    scratch = [
        pltpu.VMEM((_CHUNK,), jnp.int32),
        pltpu.VMEM((_CHUNK, D), jnp.float32),
        pltpu.SemaphoreType.DMA,
    ]

    @functools.partial(
        pl.kernel,
        out_type=jax.ShapeDtypeStruct((N, D), jnp.float32),
        mesh=mesh,
        scratch_types=scratch,
    )
    def dispatch(x_hbm, pos_hbm, xs_hbm, idx_v, rows_v, sem):
        wid = lax.axis_index("s") * _NC + lax.axis_index("c")
        base = wid * _CHUNK
        pltpu.sync_copy(pos_hbm.at[pl.ds(base, _CHUNK)], idx_v)
        pltpu.sync_copy(x_hbm.at[pl.ds(base, _CHUNK)], rows_v)
        pltpu.async_copy(rows_v, xs_hbm.at[idx_v], sem).wait()

    @functools.partial(
        pl.kernel,
        out_type=jax.ShapeDtypeStruct((N, D), jnp.float32),
        mesh=mesh,
        scratch_types=scratch,
    )
    def combine(ys_hbm, pos_hbm, out_hbm, idx_v, rows_v, sem):
        wid = lax.axis_index("s") * _NC + lax.axis_index("c")
        base = wid * _CHUNK
        pltpu.sync_copy(pos_hbm.at[pl.ds(base, _CHUNK)], idx_v)
        pltpu.async_copy(ys_hbm.at[idx_v], rows_v, sem).wait()
        pltpu.sync_copy(rows_v, out_hbm.at[pl.ds(base, _CHUNK)])

    return dispatch, combine


def _route(labels):
    labels2d = labels.reshape(_LR, _LC)
    pos2d, meta = pl.pallas_call(
        _route_body,
        out_shape=[
            jax.ShapeDtypeStruct((_LR, _LC), jnp.int32),
            jax.ShapeDtypeStruct((10, WL), jnp.int32),
        ],
    )(labels2d)
    return pos2d.reshape(N), meta


def _ffn(meta, xs, W1, b1, W2, b2):
    grid_spec = pltpu.PrefetchScalarGridSpec(
        num_scalar_prefetch=1,
        grid=(WL,),
        in_specs=[
            pl.BlockSpec((TILE, D), lambda w, m: (m[0, w], 0)),
            pl.BlockSpec((1, 1, DFF), lambda w, m: (m[1, w], 0, 0)),
            pl.BlockSpec((1, 1, D), lambda w, m: (m[1, w], 0, 0)),
            pl.BlockSpec(memory_space=pltpu.HBM),
            pl.BlockSpec(memory_space=pltpu.HBM),
        ],
        out_specs=pl.BlockSpec((TILE, D), lambda w, m: (m[0, w], 0)),
        scratch_shapes=[
            pltpu.VMEM((3, D, DFF), jnp.float32),
            pltpu.VMEM((3, DFF, D), jnp.float32),
            pltpu.SemaphoreType.DMA((2, 3)),
        ],
    )
    return pl.pallas_call(
        _ffn_body,
        grid_spec=grid_spec,
        out_shape=jax.ShapeDtypeStruct((N, D), jnp.float32),
    )(meta, xs, b1.reshape(E, 1, DFF), b2.reshape(E, 1, D), W1, W2)


def kernel(hidden_states, router_labels, W1, b1, W2, b2):
    dispatch, combine = _sc_kernels()
    pos, meta = _route(router_labels)
    xs = dispatch(hidden_states, pos)
    ys = _ffn(meta, xs, W1, b1, W2, b2)
    return combine(ys, pos)


# speedup vs baseline: 1.0310x; 1.0310x over previous
import functools

import jax
import jax.numpy as jnp
from jax import lax
from jax.experimental import pallas as pl
from jax.experimental.pallas import tpu as pltpu, tpu_sc as plsc

E = 8
D = 1024
DFF = 2048
N = 2048
TILE = 256
T = N // TILE
WL = T + E

_LR = 16
_LC = 128


def _route_body(lab_ref, pos_ref, meta_ref):
    lab = lab_ref[...]

    ci = lax.broadcasted_iota(jnp.int32, (_LC, _LC), 0)
    cj = lax.broadcasted_iota(jnp.int32, (_LC, _LC), 1)
    u_lane = (ci <= cj).astype(jnp.float32)
    ri = lax.broadcasted_iota(jnp.int32, (_LR, _LR), 0)
    rj = lax.broadcasted_iota(jnp.int32, (_LR, _LR), 1)
    l_row = (rj < ri).astype(jnp.float32)

    pos_f = jnp.zeros((_LR, _LC), jnp.float32)
    starts, ends = [], []
    run = jnp.zeros((), jnp.float32)
    for e in range(E):
        m = (lab == e).astype(jnp.float32)
        within = jnp.dot(m, u_lane, preferred_element_type=jnp.float32)
        rowsum = jnp.sum(m, axis=1, keepdims=True)
        rowpref = jnp.dot(l_row, rowsum, preferred_element_type=jnp.float32)
        rank = within + rowpref
        cnt = jnp.sum(rowsum)
        starts.append(run)
        run = run + cnt
        ends.append(run)
        pos_f = pos_f + m * (starts[e] + rank - 1.0)
    pos_ref[...] = pos_f.astype(jnp.int32)

    one = jnp.ones((), jnp.float32)
    zero = jnp.zeros((), jnp.float32)
    tlos, nspans = [], []
    woff = [zero]
    for e in range(E):
        cnt = ends[e] - starts[e]
        nonempty = (cnt > 0).astype(jnp.float32)
        tlo = jnp.floor(starts[e] / TILE)
        thi = jnp.floor((ends[e] - one) / TILE)
        n = nonempty * (thi - tlo + one)
        tlos.append(tlo)
        nspans.append(n)
        woff.append(woff[-1] + n)
    total = woff[-1]

    ts, es, los, his = [], [], [], []
    for w in range(WL):
        wf = jnp.minimum(jnp.float32(w), total - one)
        t_w = zero
        e_w = zero
        s_w = zero
        en_w = zero
        for e in range(E):
            inside = ((woff[e] <= wf) & (wf < woff[e + 1])).astype(jnp.float32)
            t_e = tlos[e] + (wf - woff[e])
            t_w = t_w + inside * t_e
            e_w = e_w + inside * e
            s_w = s_w + inside * starts[e]
            en_w = en_w + inside * ends[e]
        valid = (jnp.float32(w) < total).astype(jnp.float32)
        ts.append(t_w)
        es.append(e_w)
        los.append(valid * jnp.maximum(s_w - t_w * TILE, zero))
        his.append(valid * jnp.minimum(en_w - t_w * TILE, jnp.float32(TILE)))

    firsts, runs = [], []
    runidx = zero
    for w in range(WL):
        if w == 0:
            f = one
        else:
            f = (es[w] != es[w - 1]).astype(jnp.float32)
            runidx = runidx + f
        firsts.append(f)
        runs.append(runidx)
    nxts, hasns, nxt2s, hasn2s = [], [], [], []
    for w in range(WL):
        nxt = es[w]
        nxt2 = es[w]
        hasn = zero
        hasn2 = zero
        for v in range(WL - 1, w, -1):
            take = firsts[v]
            nxt2 = take * nxt + (one - take) * nxt2
            hasn2 = take * hasn + (one - take) * hasn2
            nxt = take * es[v] + (one - take) * nxt
            hasn = jnp.maximum(hasn, take)
        nxts.append(nxt)
        hasns.append(hasn)
        nxt2s.append(nxt2)
        hasn2s.append(hasn2)

    iota_w = lax.broadcasted_iota(jnp.int32, (1, WL), 1)
    rows_vals = [ts, es, los, his, firsts, runs, nxts, hasns, nxt2s, hasn2s]
    rows = []
    for vals in rows_vals:
        row = jnp.zeros((1, WL), jnp.float32)
        for w in range(WL):
            row = row + (iota_w == w).astype(jnp.float32) * vals[w]
        rows.append(row)
    meta_ref[...] = jnp.concatenate(rows, axis=0).astype(jnp.int32)


def _ffn_body(m_ref, x_ref, b1_ref, b2_ref, w1_hbm, w2_hbm, o_ref,
              w1_buf, w2_buf, sems):
    w = pl.program_id(0)
    e = m_ref[1, w]
    lo = m_ref[2, w]
    hi = m_ref[3, w]
    first = m_ref[4, w]
    run = m_ref[5, w]
    nxt = m_ref[6, w]
    hasn = m_ref[7, w]
    slot = lax.rem(run, 2)
    nslot = lax.rem(run + 1, 2)

    def w_copies(exp, s):
        return (
            pltpu.make_async_copy(w1_hbm.at[exp], w1_buf.at[s], sems.at[0, s]),
            pltpu.make_async_copy(w2_hbm.at[exp], w2_buf.at[s], sems.at[1, s]),
        )

    @pl.when(w == 0)
    def _():
        for c in w_copies(e, slot):
            c.start()

    @pl.when(first == 1)
    def _():
        @pl.when(hasn == 1)
        def _():
            for c in w_copies(nxt, nslot):
                c.start()

        for c in w_copies(e, slot):
            c.wait()

    @pl.when(hi > lo)
    def _():
        x = x_ref[...]
        h = jnp.dot(x, w1_buf[slot], preferred_element_type=jnp.float32)
        h = h + b1_ref[0]
        h = jax.nn.gelu(h)
        y = jnp.dot(h, w2_buf[slot], preferred_element_type=jnp.float32)
        y = y + b2_ref[0]
        rows = lax.broadcasted_iota(jnp.int32, (TILE, 1), 0)
        mask = (rows >= lo) & (rows < hi)
        o_ref[...] = jnp.where(mask, y, o_ref[...])


_NC = 2
_NW = 32
_CHUNK = N // _NW


@functools.lru_cache(maxsize=None)
def _sc_kernels():
    mesh = plsc.VectorSubcoreMesh(core_axis_name="c", subcore_axis_name="s")
    scratch = [
        pltpu.VMEM((_CHUNK,), jnp.int32),
        pltpu.VMEM((_CHUNK, D), jnp.float32),
        pltpu.SemaphoreType.DMA,
    ]

    @functools.partial(
        pl.kernel,
        out_type=jax.ShapeDtypeStruct((N, D), jnp.float32),
        mesh=mesh,
        scratch_types=scratch,
    )
    def dispatch(x_hbm, pos_hbm, xs_hbm, idx_v, rows_v, sem):
        wid = lax.axis_index("s") * _NC + lax.axis_index("c")
        base = wid * _CHUNK
        pltpu.sync_copy(pos_hbm.at[pl.ds(base, _CHUNK)], idx_v)
        pltpu.sync_copy(x_hbm.at[pl.ds(base, _CHUNK)], rows_v)
        pltpu.async_copy(rows_v, xs_hbm.at[idx_v], sem).wait()

    @functools.partial(
        pl.kernel,
        out_type=jax.ShapeDtypeStruct((N, D), jnp.float32),
        mesh=mesh,
        scratch_types=scratch,
    )
    def combine(ys_hbm, pos_hbm, out_hbm, idx_v, rows_v, sem):
        wid = lax.axis_index("s") * _NC + lax.axis_index("c")
        base = wid * _CHUNK
        pltpu.sync_copy(pos_hbm.at[pl.ds(base, _CHUNK)], idx_v)
        pltpu.async_copy(ys_hbm.at[idx_v], rows_v, sem).wait()
        pltpu.sync_copy(rows_v, out_hbm.at[pl.ds(base, _CHUNK)])

    return dispatch, combine


def _route(labels):
    labels2d = labels.reshape(_LR, _LC)
    pos2d, meta = pl.pallas_call(
        _route_body,
        out_shape=[
            jax.ShapeDtypeStruct((_LR, _LC), jnp.int32),
            jax.ShapeDtypeStruct((10, WL), jnp.int32),
        ],
    )(labels2d)
    return pos2d.reshape(N), meta


def _ffn(meta, xs, W1, b1, W2, b2):
    grid_spec = pltpu.PrefetchScalarGridSpec(
        num_scalar_prefetch=1,
        grid=(WL,),
        in_specs=[
            pl.BlockSpec((TILE, D), lambda w, m: (m[0, w], 0)),
            pl.BlockSpec((1, 1, DFF), lambda w, m: (m[1, w], 0, 0)),
            pl.BlockSpec((1, 1, D), lambda w, m: (m[1, w], 0, 0)),
            pl.BlockSpec(memory_space=pltpu.HBM),
            pl.BlockSpec(memory_space=pltpu.HBM),
        ],
        out_specs=pl.BlockSpec((TILE, D), lambda w, m: (m[0, w], 0)),
        scratch_shapes=[
            pltpu.VMEM((2, D, DFF), jnp.float32),
            pltpu.VMEM((2, DFF, D), jnp.float32),
            pltpu.SemaphoreType.DMA((2, 2)),
        ],
    )
    return pl.pallas_call(
        _ffn_body,
        grid_spec=grid_spec,
        out_shape=jax.ShapeDtypeStruct((N, D), jnp.float32),
    )(meta, xs, b1.reshape(E, 1, DFF), b2.reshape(E, 1, D), W1, W2)


def kernel(hidden_states, router_labels, W1, b1, W2, b2):
    dispatch, combine = _sc_kernels()
    pos, meta = _route(router_labels)
    xs = dispatch(hidden_states, pos)
    ys = _ffn(meta, xs, W1, b1, W2, b2)
    return combine(ys, pos)


# speedup vs baseline: 1.0422x; 1.0109x over previous
import functools

import jax
import jax.numpy as jnp
from jax import lax
from jax.experimental import pallas as pl
from jax.experimental.pallas import tpu as pltpu, tpu_sc as plsc

E = 8
D = 1024
DFF = 2048
N = 2048
TILE = 256
T = N // TILE
WL = T + E

_LR = 16
_LC = 128


def _route_body(lab_ref, pos_ref, meta_ref):
    lab = lab_ref[...]

    ci = lax.broadcasted_iota(jnp.int32, (_LC, _LC), 0)
    cj = lax.broadcasted_iota(jnp.int32, (_LC, _LC), 1)
    u_lane = (ci <= cj).astype(jnp.float32)
    ri = lax.broadcasted_iota(jnp.int32, (_LR, _LR), 0)
    rj = lax.broadcasted_iota(jnp.int32, (_LR, _LR), 1)
    l_row = (rj < ri).astype(jnp.float32)

    pos_f = jnp.zeros((_LR, _LC), jnp.float32)
    starts, ends = [], []
    run = jnp.zeros((), jnp.float32)
    for e in range(E):
        m = (lab == e).astype(jnp.float32)
        within = jnp.dot(m, u_lane, preferred_element_type=jnp.float32)
        rowsum = jnp.sum(m, axis=1, keepdims=True)
        rowpref = jnp.dot(l_row, rowsum, preferred_element_type=jnp.float32)
        rank = within + rowpref
        cnt = jnp.sum(rowsum)
        starts.append(run)
        run = run + cnt
        ends.append(run)
        pos_f = pos_f + m * (starts[e] + rank - 1.0)
    pos_ref[...] = pos_f.astype(jnp.int32)

    one = jnp.ones((), jnp.float32)
    zero = jnp.zeros((), jnp.float32)
    tlos, nspans = [], []
    woff = [zero]
    for e in range(E):
        cnt = ends[e] - starts[e]
        nonempty = (cnt > 0).astype(jnp.float32)
        tlo = jnp.floor(starts[e] / TILE)
        thi = jnp.floor((ends[e] - one) / TILE)
        n = nonempty * (thi - tlo + one)
        tlos.append(tlo)
        nspans.append(n)
        woff.append(woff[-1] + n)
    total = woff[-1]

    ts, es, los, his = [], [], [], []
    for w in range(WL):
        wf = jnp.minimum(jnp.float32(w), total - one)
        t_w = zero
        e_w = zero
        s_w = zero
        en_w = zero
        for e in range(E):
            inside = ((woff[e] <= wf) & (wf < woff[e + 1])).astype(jnp.float32)
            t_e = tlos[e] + (wf - woff[e])
            t_w = t_w + inside * t_e
            e_w = e_w + inside * e
            s_w = s_w + inside * starts[e]
            en_w = en_w + inside * ends[e]
        valid = (jnp.float32(w) < total).astype(jnp.float32)
        ts.append(t_w)
        es.append(e_w)
        los.append(valid * jnp.maximum(s_w - t_w * TILE, zero))
        his.append(valid * jnp.minimum(en_w - t_w * TILE, jnp.float32(TILE)))

    firsts, runs = [], []
    runidx = zero
    for w in range(WL):
        if w == 0:
            f = one
        else:
            f = (es[w] != es[w - 1]).astype(jnp.float32)
            runidx = runidx + f
        firsts.append(f)
        runs.append(runidx)
    nxts, hasns, nxt2s, hasn2s = [], [], [], []
    for w in range(WL):
        nxt = es[w]
        nxt2 = es[w]
        hasn = zero
        hasn2 = zero
        for v in range(WL - 1, w, -1):
            take = firsts[v]
            nxt2 = take * nxt + (one - take) * nxt2
            hasn2 = take * hasn + (one - take) * hasn2
            nxt = take * es[v] + (one - take) * nxt
            hasn = jnp.maximum(hasn, take)
        nxts.append(nxt)
        hasns.append(hasn)
        nxt2s.append(nxt2)
        hasn2s.append(hasn2)

    iota_w = lax.broadcasted_iota(jnp.int32, (1, WL), 1)
    rows_vals = [ts, es, los, his, firsts, runs, nxts, hasns, nxt2s, hasn2s]
    rows = []
    for vals in rows_vals:
        row = jnp.zeros((1, WL), jnp.float32)
        for w in range(WL):
            row = row + (iota_w == w).astype(jnp.float32) * vals[w]
        rows.append(row)
    meta_ref[...] = jnp.concatenate(rows, axis=0).astype(jnp.int32)


def _ffn_body(m_ref, x_ref, b1_ref, b2_ref, w1_hbm, w2_hbm, o_ref,
              w1_buf, w2_buf, sems):
    w = pl.program_id(0)
    e = m_ref[1, w]
    lo = m_ref[2, w]
    hi = m_ref[3, w]
    first = m_ref[4, w]
    run = m_ref[5, w]
    nxt = m_ref[6, w]
    hasn = m_ref[7, w]
    slot = lax.rem(run, 2)
    nslot = lax.rem(run + 1, 2)

    def w_copies(exp, s):
        return (
            pltpu.make_async_copy(w1_hbm.at[exp], w1_buf.at[s], sems.at[0, s]),
            pltpu.make_async_copy(w2_hbm.at[exp], w2_buf.at[s], sems.at[1, s]),
        )

    @pl.when(w == 0)
    def _():
        for c in w_copies(e, slot):
            c.start()

    @pl.when(first == 1)
    def _():
        @pl.when(hasn == 1)
        def _():
            for c in w_copies(nxt, nslot):
                c.start()

        for c in w_copies(e, slot):
            c.wait()

    @pl.when(hi > lo)
    def _():
        x = x_ref[...]
        h = jnp.dot(x, w1_buf[slot], preferred_element_type=jnp.float32)
        h = h + b1_ref[0]
        h = jax.nn.gelu(h)
        y = jnp.dot(h, w2_buf[slot], preferred_element_type=jnp.float32)
        y = y + b2_ref[0]
        rows = lax.broadcasted_iota(jnp.int32, (TILE, 1), 0)
        mask = (rows >= lo) & (rows < hi)
        o_ref[...] = jnp.where(mask, y, o_ref[...])


_NC = 2
_NW = 32
_CHUNK = N // _NW


_SCC = 2
_CR = _CHUNK // _SCC


@functools.lru_cache(maxsize=None)
def _sc_kernels():
    mesh = plsc.VectorSubcoreMesh(core_axis_name="c", subcore_axis_name="s")
    scratch = [
        pltpu.VMEM((_SCC, _CR), jnp.int32),
        pltpu.VMEM((_SCC, _CR, D), jnp.float32),
        pltpu.SemaphoreType.DMA((_SCC,)),
        pltpu.SemaphoreType.DMA((_SCC,)),
    ]

    @functools.partial(
        pl.kernel,
        out_type=jax.ShapeDtypeStruct((N, D), jnp.float32),
        mesh=mesh,
        scratch_types=scratch,
    )
    def dispatch(x_hbm, pos_hbm, xs_hbm, idx_v, rows_v, in_sems, out_sems):
        wid = lax.axis_index("s") * _NC + lax.axis_index("c")
        base = wid * _CHUNK
        loads = []
        for k in range(_SCC):
            pltpu.sync_copy(pos_hbm.at[pl.ds(base + k * _CR, _CR)],
                            idx_v.at[k])
            c = pltpu.make_async_copy(x_hbm.at[pl.ds(base + k * _CR, _CR)],
                                      rows_v.at[k], in_sems.at[k])
            c.start()
            loads.append(c)
        scatters = []
        for k in range(_SCC):
            loads[k].wait()
            c = pltpu.make_async_copy(rows_v.at[k], xs_hbm.at[idx_v.at[k]],
                                      out_sems.at[k])
            c.start()
            scatters.append(c)
        for c in scatters:
            c.wait()

    @functools.partial(
        pl.kernel,
        out_type=jax.ShapeDtypeStruct((N, D), jnp.float32),
        mesh=mesh,
        scratch_types=scratch,
    )
    def combine(ys_hbm, pos_hbm, out_hbm, idx_v, rows_v, in_sems, out_sems):
        wid = lax.axis_index("s") * _NC + lax.axis_index("c")
        base = wid * _CHUNK
        gathers = []
        for k in range(_SCC):
            pltpu.sync_copy(pos_hbm.at[pl.ds(base + k * _CR, _CR)],
                            idx_v.at[k])
            c = pltpu.make_async_copy(ys_hbm.at[idx_v.at[k]], rows_v.at[k],
                                      in_sems.at[k])
            c.start()
            gathers.append(c)
        stores = []
        for k in range(_SCC):
            gathers[k].wait()
            c = pltpu.make_async_copy(rows_v.at[k],
                                      out_hbm.at[pl.ds(base + k * _CR, _CR)],
                                      out_sems.at[k])
            c.start()
            stores.append(c)
        for c in stores:
            c.wait()

    return dispatch, combine


def _route(labels):
    labels2d = labels.reshape(_LR, _LC)
    pos2d, meta = pl.pallas_call(
        _route_body,
        out_shape=[
            jax.ShapeDtypeStruct((_LR, _LC), jnp.int32),
            jax.ShapeDtypeStruct((10, WL), jnp.int32),
        ],
    )(labels2d)
    return pos2d.reshape(N), meta


def _ffn(meta, xs, W1, b1, W2, b2):
    grid_spec = pltpu.PrefetchScalarGridSpec(
        num_scalar_prefetch=1,
        grid=(WL,),
        in_specs=[
            pl.BlockSpec((TILE, D), lambda w, m: (m[0, w], 0)),
            pl.BlockSpec((1, 1, DFF), lambda w, m: (m[1, w], 0, 0)),
            pl.BlockSpec((1, 1, D), lambda w, m: (m[1, w], 0, 0)),
            pl.BlockSpec(memory_space=pltpu.HBM),
            pl.BlockSpec(memory_space=pltpu.HBM),
        ],
        out_specs=pl.BlockSpec((TILE, D), lambda w, m: (m[0, w], 0)),
        scratch_shapes=[
            pltpu.VMEM((2, D, DFF), jnp.float32),
            pltpu.VMEM((2, DFF, D), jnp.float32),
            pltpu.SemaphoreType.DMA((2, 2)),
        ],
    )
    return pl.pallas_call(
        _ffn_body,
        grid_spec=grid_spec,
        out_shape=jax.ShapeDtypeStruct((N, D), jnp.float32),
    )(meta, xs, b1.reshape(E, 1, DFF), b2.reshape(E, 1, D), W1, W2)


def kernel(hidden_states, router_labels, W1, b1, W2, b2):
    dispatch, combine = _sc_kernels()
    pos, meta = _route(router_labels)
    xs = dispatch(hidden_states, pos)
    ys = _ffn(meta, xs, W1, b1, W2, b2)
    return combine(ys, pos)
